# baseline (speedup 1.0000x reference)
import functools

import jax
import jax.numpy as jnp
from jax import lax
from jax.experimental import pallas as pl
from jax.experimental.pallas import tpu as pltpu
from jax.experimental.pallas import tpu_sc as plsc

B, N, D = 4096, 200, 128
NC, NS, L = 2, 16, 16
NW = NC * NS
HALVES = 2
HB = B // HALVES
BPW = HB // NW
BT = 1024


def _sc_gather_body(table, idx1_hbm, idx2_hbm, base_hbm, ll1_hbm, ll2_hbm,
                    idx_v1, idx_v2, base_v, rows1, rows2,
                    sem1, sem2, sem3, sem4):
    wid = lax.axis_index("s") * NC + lax.axis_index("c")
    base = wid * BPW
    ci1 = pltpu.async_copy(idx1_hbm.at[pl.ds(base, BPW)], idx_v1, sem1)
    ci2 = pltpu.async_copy(idx2_hbm.at[pl.ds(base, BPW)], idx_v2, sem2)
    cb = pltpu.async_copy(base_hbm.at[pl.ds(base, BPW)], base_v, sem3)
    ci1.wait()
    ci2.wait()
    cb.wait()
    for i in range(BPW // L):
        sl = pl.ds(i * L, L)
        off = base_v[sl]
        idx_v1[sl] = idx_v1[sl] + off
        idx_v2[sl] = idx_v2[sl] + off
    c1 = pltpu.async_copy(table.at[idx_v1], rows1, sem1)
    c2 = pltpu.async_copy(table.at[idx_v2], rows2, sem2)
    c1.wait()
    o1 = pltpu.async_copy(rows1, ll1_hbm.at[pl.ds(base, BPW)], sem3)
    c2.wait()
    o2 = pltpu.async_copy(rows2, ll2_hbm.at[pl.ds(base, BPW)], sem4)
    o1.wait()
    o2.wait()


@functools.cache
def _sc_gather():
    return pl.kernel(
        _sc_gather_body,
        out_type=(
            jax.ShapeDtypeStruct((HB, D), jnp.float32),
            jax.ShapeDtypeStruct((HB, D), jnp.float32),
        ),
        mesh=plsc.VectorSubcoreMesh(
            core_axis_name="c", subcore_axis_name="s",
            num_cores=NC, num_subcores=NS,
        ),
        scratch_types=[
            pltpu.VMEM((BPW,), jnp.int32),
            pltpu.VMEM((BPW,), jnp.int32),
            pltpu.VMEM((BPW,), jnp.int32),
            pltpu.VMEM((BPW, D), jnp.float32),
            pltpu.VMEM((BPW, D), jnp.float32),
            pltpu.SemaphoreType.DMA,
            pltpu.SemaphoreType.DMA,
            pltpu.SemaphoreType.DMA,
            pltpu.SemaphoreType.DMA,
        ],
    )


def _tc_linear_body(ll1_ref, ll2_ref, r1_ref, r2_ref, wr_ref, wl_ref, out_ref):
    wf = jnp.dot(wr_ref[...], wl_ref[2 * D:, :],
                 preferred_element_type=jnp.float32)
    ratio = r1_ref[...] / r2_ref[...]
    acc = jnp.dot(ll1_ref[...], wl_ref[:D, :],
                  preferred_element_type=jnp.float32)
    acc = acc + jnp.dot(ll2_ref[...], wl_ref[D:2 * D, :],
                        preferred_element_type=jnp.float32)
    out_ref[...] = acc + ratio * wf


_tc_linear = pl.pallas_call(
    _tc_linear_body,
    grid=(HB // BT,),
    in_specs=[
        pl.BlockSpec((BT, D), lambda i: (i, 0)),
        pl.BlockSpec((BT, D), lambda i: (i, 0)),
        pl.BlockSpec((BT, 1), lambda i: (i, 0)),
        pl.BlockSpec((BT, 1), lambda i: (i, 0)),
        pl.BlockSpec((1, D), lambda i: (0, 0)),
        pl.BlockSpec((3 * D, D), lambda i: (0, 0)),
    ],
    out_specs=pl.BlockSpec((BT, D), lambda i: (i, 0)),
    out_shape=jax.ShapeDtypeStruct((HB, D), jnp.float32),
)


def kernel(encoded_row, loadlock1_wafer_in, loadlock2_wafer_in, W_ratio,
           W_lin, loadlock1_wafer_recipe, loadlock2_wafer_recipe):
    table = encoded_row.reshape(B * N, D)
    row_base = jnp.arange(B, dtype=jnp.int32) * N
    sc = _sc_gather()
    gathered = []
    for h in range(HALVES):
        s = slice(h * HB, (h + 1) * HB)
        gathered.append(sc(table, loadlock1_wafer_recipe[s],
                           loadlock2_wafer_recipe[s], row_base[s]))
    outs = []
    for h in range(HALVES):
        s = slice(h * HB, (h + 1) * HB)
        ll1, ll2 = gathered[h]
        outs.append(_tc_linear(ll1, ll2, loadlock1_wafer_in[s],
                               loadlock2_wafer_in[s], W_ratio, W_lin))
    return jnp.concatenate(outs, axis=0)


# speedup vs baseline: 1.1619x; 1.1619x over previous
import functools

import jax
import jax.numpy as jnp
from jax import lax
from jax.experimental import pallas as pl
from jax.experimental.pallas import tpu as pltpu
from jax.experimental.pallas import tpu_sc as plsc

B, N, D = 4096, 200, 128
NC, NS, L = 2, 16, 16
NW = NC * NS
BPW = B // NW
C = 4
CH = BPW // C
BT = 1024


def _sc_gather_body(table, idx1_hbm, idx2_hbm, ll1_hbm, ll2_hbm,
                    idx_v1, idx_v2, rows1, rows2,
                    semi1, semi2, semg1, semg2, semo1, semo2):
    wid = lax.axis_index("s") * NC + lax.axis_index("c")
    base = wid * BPW
    ci1 = pltpu.async_copy(idx1_hbm.at[wid], idx_v1, semi1)
    ci2 = pltpu.async_copy(idx2_hbm.at[wid], idx_v2, semi2)
    ci1.wait()
    ci2.wait()
    lane = lax.iota(jnp.int32, L) * N
    for c in range(C):
        for j in range(CH // L):
            off = lane + (base + c * CH + j * L) * N
            sl = pl.ds(j * L, L)
            idx_v1[c, sl] = idx_v1[c, sl] + off
            idx_v2[c, sl] = idx_v2[c, sl] + off
    gs = []
    for c in range(C):
        g1 = pltpu.async_copy(table.at[idx_v1.at[c]], rows1.at[c], semg1)
        g2 = pltpu.async_copy(table.at[idx_v2.at[c]], rows2.at[c], semg2)
        gs.append((g1, g2))
    outs = []
    for c in range(C):
        g1, g2 = gs[c]
        dst = pl.ds(base + c * CH, CH)
        g1.wait()
        outs.append(pltpu.async_copy(rows1.at[c], ll1_hbm.at[dst], semo1))
        g2.wait()
        outs.append(pltpu.async_copy(rows2.at[c], ll2_hbm.at[dst], semo2))
    for o in outs:
        o.wait()


@functools.cache
def _sc_gather():
    return pl.kernel(
        _sc_gather_body,
        out_type=(
            jax.ShapeDtypeStruct((B, D), jnp.float32),
            jax.ShapeDtypeStruct((B, D), jnp.float32),
        ),
        mesh=plsc.VectorSubcoreMesh(
            core_axis_name="c", subcore_axis_name="s",
            num_cores=NC, num_subcores=NS,
        ),
        scratch_types=[
            pltpu.VMEM((C, CH), jnp.int32),
            pltpu.VMEM((C, CH), jnp.int32),
            pltpu.VMEM((C, CH, D), jnp.float32),
            pltpu.VMEM((C, CH, D), jnp.float32),
            pltpu.SemaphoreType.DMA,
            pltpu.SemaphoreType.DMA,
            pltpu.SemaphoreType.DMA,
            pltpu.SemaphoreType.DMA,
            pltpu.SemaphoreType.DMA,
            pltpu.SemaphoreType.DMA,
        ],
    )


def _tc_linear_body(ll1_ref, ll2_ref, r1_ref, r2_ref, wr_ref, wl_ref, out_ref):
    wf = jnp.dot(wr_ref[...], wl_ref[2 * D:, :],
                 preferred_element_type=jnp.float32)
    ratio = r1_ref[...] / r2_ref[...]
    acc = jnp.dot(ll1_ref[...], wl_ref[:D, :],
                  preferred_element_type=jnp.float32)
    acc = acc + jnp.dot(ll2_ref[...], wl_ref[D:2 * D, :],
                        preferred_element_type=jnp.float32)
    out_ref[...] = acc + ratio * wf


_tc_linear = pl.pallas_call(
    _tc_linear_body,
    grid=(B // BT,),
    in_specs=[
        pl.BlockSpec((BT, D), lambda i: (i, 0)),
        pl.BlockSpec((BT, D), lambda i: (i, 0)),
        pl.BlockSpec((BT, 1), lambda i: (i, 0)),
        pl.BlockSpec((BT, 1), lambda i: (i, 0)),
        pl.BlockSpec((1, D), lambda i: (0, 0)),
        pl.BlockSpec((3 * D, D), lambda i: (0, 0)),
    ],
    out_specs=pl.BlockSpec((BT, D), lambda i: (i, 0)),
    out_shape=jax.ShapeDtypeStruct((B, D), jnp.float32),
)


def kernel(encoded_row, loadlock1_wafer_in, loadlock2_wafer_in, W_ratio,
           W_lin, loadlock1_wafer_recipe, loadlock2_wafer_recipe):
    table = encoded_row.reshape(B * N, D)
    idx1 = loadlock1_wafer_recipe.reshape(NW, C, CH)
    idx2 = loadlock2_wafer_recipe.reshape(NW, C, CH)
    ll1, ll2 = _sc_gather()(table, idx1, idx2)
    return _tc_linear(ll1, ll2, loadlock1_wafer_in, loadlock2_wafer_in,
                      W_ratio, W_lin)


# speedup vs baseline: 1.2525x; 1.0779x over previous
import functools

import jax
import jax.numpy as jnp
from jax import lax
from jax.experimental import pallas as pl
from jax.experimental.pallas import tpu as pltpu
from jax.experimental.pallas import tpu_sc as plsc

B, N, D = 4096, 200, 128
NC, NS, L = 2, 16, 16
NW = NC * NS
BPW = B // NW
BT = 2048


def _sc_gather_body(table, idx1_hbm, idx2_hbm, ll1_hbm, ll2_hbm,
                    idx_v1, idx_v2, rows1, rows2, sem1, sem2, sem3, sem4):
    wid = lax.axis_index("s") * NC + lax.axis_index("c")
    base = wid * BPW
    ci1 = pltpu.async_copy(idx1_hbm.at[pl.ds(base, BPW)], idx_v1, sem1)
    ci2 = pltpu.async_copy(idx2_hbm.at[pl.ds(base, BPW)], idx_v2, sem2)
    ci1.wait()
    ci2.wait()
    lane = lax.iota(jnp.int32, L) * N
    for i in range(BPW // L):
        off = lane + (base + i * L) * N
        sl = pl.ds(i * L, L)
        idx_v1[sl] = idx_v1[sl] + off
        idx_v2[sl] = idx_v2[sl] + off
    c1 = pltpu.async_copy(table.at[idx_v1], rows1, sem1)
    c2 = pltpu.async_copy(table.at[idx_v2], rows2, sem2)
    c1.wait()
    o1 = pltpu.async_copy(rows1, ll1_hbm.at[pl.ds(base, BPW)], sem3)
    c2.wait()
    o2 = pltpu.async_copy(rows2, ll2_hbm.at[pl.ds(base, BPW)], sem4)
    o1.wait()
    o2.wait()


@functools.cache
def _sc_gather():
    return pl.kernel(
        _sc_gather_body,
        out_type=(
            jax.ShapeDtypeStruct((B, D), jnp.float32),
            jax.ShapeDtypeStruct((B, D), jnp.float32),
        ),
        mesh=plsc.VectorSubcoreMesh(
            core_axis_name="c", subcore_axis_name="s",
            num_cores=NC, num_subcores=NS,
        ),
        scratch_types=[
            pltpu.VMEM((BPW,), jnp.int32),
            pltpu.VMEM((BPW,), jnp.int32),
            pltpu.VMEM((BPW, D), jnp.float32),
            pltpu.VMEM((BPW, D), jnp.float32),
            pltpu.SemaphoreType.DMA,
            pltpu.SemaphoreType.DMA,
            pltpu.SemaphoreType.DMA,
            pltpu.SemaphoreType.DMA,
        ],
    )


def _tc_linear_body(ll1_ref, ll2_ref, r1_ref, r2_ref, wr_ref, wl_ref, out_ref):
    wf = jnp.dot(wr_ref[...], wl_ref[2 * D:, :],
                 preferred_element_type=jnp.float32)
    ratio = r1_ref[...] / r2_ref[...]
    acc = jnp.dot(ll1_ref[...], wl_ref[:D, :],
                  preferred_element_type=jnp.float32)
    acc = acc + jnp.dot(ll2_ref[...], wl_ref[D:2 * D, :],
                        preferred_element_type=jnp.float32)
    out_ref[...] = acc + ratio * wf


_tc_linear = pl.pallas_call(
    _tc_linear_body,
    grid=(B // BT,),
    in_specs=[
        pl.BlockSpec((BT, D), lambda i: (i, 0)),
        pl.BlockSpec((BT, D), lambda i: (i, 0)),
        pl.BlockSpec((BT, 1), lambda i: (i, 0)),
        pl.BlockSpec((BT, 1), lambda i: (i, 0)),
        pl.BlockSpec((1, D), lambda i: (0, 0)),
        pl.BlockSpec((3 * D, D), lambda i: (0, 0)),
    ],
    out_specs=pl.BlockSpec((BT, D), lambda i: (i, 0)),
    out_shape=jax.ShapeDtypeStruct((B, D), jnp.float32),
)


def kernel(encoded_row, loadlock1_wafer_in, loadlock2_wafer_in, W_ratio,
           W_lin, loadlock1_wafer_recipe, loadlock2_wafer_recipe):
    table = encoded_row.reshape(B * N, D)
    ll1, ll2 = _sc_gather()(table, loadlock1_wafer_recipe,
                            loadlock2_wafer_recipe)
    return _tc_linear(ll1, ll2, loadlock1_wafer_in, loadlock2_wafer_in,
                      W_ratio, W_lin)
